# baseline (speedup 1.0000x reference)
import functools

import jax
import jax.numpy as jnp
import numpy as np
from jax.experimental import pallas as pl
from jax.experimental.pallas import tpu as pltpu

N = 4096
DIN = 128
H = 128
C = 16
L = 8

WB = 8
SGRID = N // WB
BN = 512

TINY = float(np.finfo(np.float32).tiny)

_FOLDED_KEYS = (
    (0x6D3E048F, 0x1022172D),
    (0x03D7B32D, 0xADD083F4),
    (0x92FB20EA, 0x0F38D913),
    (0xBAD56946, 0x354BA891),
    (0xB013AEE3, 0xC34EDDF6),
    (0xA4D91A96, 0x3122544E),
    (0xA506C508, 0xB6207291),
)


def _u32(v):
    return jnp.uint32(v & 0xFFFFFFFF)


def _threefry_bits(k0, k1, x1):
    ks0 = _u32(k0)
    ks1 = _u32(k1)
    ks2 = ks0 ^ ks1 ^ _u32(0x1BD11BDA)
    ks = (ks0, ks1, ks2)
    rotations = ((13, 15, 26, 6), (17, 29, 16, 24))
    x0 = jnp.full_like(x1, ks0)
    x1 = x1 + ks1
    for i in range(5):
        for r in rotations[i % 2]:
            x0 = x0 + x1
            x1 = (x1 << r) | (x1 >> (32 - r))
            x1 = x0 ^ x1
        x0 = x0 + ks[(i + 1) % 3]
        x1 = x1 + ks[(i + 2) % 3] + _u32(i + 1)
    return x0 ^ x1


def _gumbel_from_bits(bits):
    fb = (bits >> 9) | _u32(0x3F800000)
    f = jax.lax.bitcast_convert_type(fb, jnp.float32) - jnp.float32(1.0)
    one_minus_tiny = jnp.float32(np.float32(1.0) - np.float32(TINY))
    u = jnp.maximum(jnp.float32(TINY), f * one_minus_tiny + jnp.float32(TINY))
    return -jnp.log(-jnp.log(u))


def _sample_body(k0, k1, cur_ref, *refs):
    row_refs = refs[0:WB]
    inv_ref = refs[WB]
    eq_ref = refs[WB + 1]
    nxt_ref = refs[WB + 2]
    invcol_ref = refs[WB + 3]
    eqcol_ref = refs[WB + 4]

    i = pl.program_id(0)

    rows = jnp.concatenate([r[0] for r in row_refs], axis=0)
    s = jnp.sum(rows, axis=1, keepdims=True)
    logit = jnp.log(rows / s + jnp.float32(1e-9))

    col_u = jax.lax.broadcasted_iota(jnp.uint32, (WB, N), 1)
    walk_u = jnp.uint32(i * WB) + jax.lax.broadcasted_iota(jnp.uint32, (WB, N), 0)
    t = walk_u * jnp.uint32(N) + col_u
    g = _gumbel_from_bits(_threefry_bits(k0, k1, t))

    z = logit + g
    m = jnp.max(z, axis=1, keepdims=True)
    col_i = jax.lax.broadcasted_iota(jnp.int32, (WB, N), 1)
    idx = jnp.min(jnp.where(z >= m, col_i, jnp.int32(N)), axis=1)

    nxt_ref[0, 0, :] = idx
    for k in range(WB):
        ik = idx[k]
        invcol_ref[pl.ds(k, 1), :] = inv_ref[pl.ds(ik, 1), :]
        eqcol_ref[pl.ds(k, 1), :] = eq_ref[pl.ds(ik, 1), :]


def _make_sample_call(step):
    k0, k1 = _FOLDED_KEYS[step]
    body = functools.partial(_sample_body, k0, k1)
    row_spec = lambda k: pl.BlockSpec(
        (1, 1, N), lambda i, c, k=k: (c[i * WB + k], 0, 0))
    grid_spec = pltpu.PrefetchScalarGridSpec(
        num_scalar_prefetch=1,
        grid=(SGRID,),
        in_specs=[row_spec(k) for k in range(WB)]
        + [
            pl.BlockSpec((N, DIN), lambda i, c: (0, 0)),
            pl.BlockSpec((N, 3), lambda i, c: (0, 0)),
        ],
        out_specs=[
            pl.BlockSpec((1, 1, WB), lambda i, c: (i, 0, 0)),
            pl.BlockSpec((WB, DIN), lambda i, c: (i, 0)),
            pl.BlockSpec((WB, 3), lambda i, c: (i, 0)),
        ],
    )
    return pl.pallas_call(
        body,
        grid_spec=grid_spec,
        out_shape=[
            jax.ShapeDtypeStruct((SGRID, 1, WB), jnp.int32),
            jax.ShapeDtypeStruct((N, DIN), jnp.float32),
            jax.ShapeDtypeStruct((N, 3), jnp.float32),
        ],
    )


def _recur_body(*refs):
    inv_refs = refs[0:L]
    eq_refs = refs[L:2 * L]
    w_in_ref, w_h_ref, b_ref, w_gate_ref, w_mix_ref = refs[2 * L:2 * L + 5]
    inv_traj_ref, eq_traj_ref = refs[2 * L + 5:]

    pooled_inv = inv_refs[0][...]
    for r in inv_refs[1:]:
        pooled_inv = pooled_inv + r[...]
    pooled_inv = pooled_inv * jnp.float32(1.0 / L)

    pooled_eq = eq_refs[0][...]
    for r in eq_refs[1:]:
        pooled_eq = pooled_eq + r[...]
    pooled_eq = pooled_eq * jnp.float32(1.0 / L)

    w_in = w_in_ref[...]
    w_h = w_h_ref[...]
    b = b_ref[...]
    w_gate = w_gate_ref[...]
    w_mix = w_mix_ref[...]

    a = jnp.dot(pooled_inv, w_in, preferred_element_type=jnp.float32) + b
    src = [pooled_eq[:, d:d + 1] * w_mix for d in range(3)]

    inv_h = jnp.zeros((BN, H), jnp.float32)
    eq_h = [jnp.zeros((BN, C), jnp.float32) for _ in range(3)]
    for step in range(L):
        inv_h = jnp.tanh(
            a + jnp.dot(inv_h, w_h, preferred_element_type=jnp.float32))
        gate = jax.nn.sigmoid(
            jnp.dot(inv_h, w_gate, preferred_element_type=jnp.float32))
        inv_traj_ref[step] = inv_h
        for d in range(3):
            eq_h[d] = eq_h[d] * gate + src[d]
            eq_traj_ref[step, :, d, :] = eq_h[d]


def _recur_call():
    nb = N // BN
    full2 = lambda shape: pl.BlockSpec(shape, lambda i: (0, 0))
    return pl.pallas_call(
        _recur_body,
        grid=(nb,),
        in_specs=[pl.BlockSpec((BN, DIN), lambda i: (i, 0)) for _ in range(L)]
        + [pl.BlockSpec((BN, 3), lambda i: (i, 0)) for _ in range(L)]
        + [
            full2((DIN, H)),
            full2((H, H)),
            full2((1, H)),
            full2((H, C)),
            full2((1, C)),
        ],
        out_specs=[
            pl.BlockSpec((L, BN, H), lambda i: (0, i, 0)),
            pl.BlockSpec((L, BN, 3, C), lambda i: (0, i, 0, 0)),
        ],
        out_shape=[
            jax.ShapeDtypeStruct((L, N, H), jnp.float32),
            jax.ShapeDtypeStruct((L, N, 3, C), jnp.float32),
        ],
    )


@jax.jit
def kernel(probability, invariant_input, equivariant_input, W_in, W_h, b,
           W_gate, w_mix):
    cur = jnp.arange(N, dtype=jnp.int32)
    prob3 = probability.reshape(N, 1, N)
    inv_cols = [invariant_input]
    eq_cols = [equivariant_input]
    for step in range(L - 1):
        nxt, invcol, eqcol = _make_sample_call(step)(
            cur, prob3, prob3, prob3, prob3, prob3, prob3, prob3, prob3,
            invariant_input, equivariant_input)
        cur = nxt.reshape(N)
        inv_cols.append(invcol)
        eq_cols.append(eqcol)

    inv_traj, eq_traj = _recur_call()(
        *inv_cols, *eq_cols, W_in, W_h, b.reshape(1, H), W_gate,
        w_mix.reshape(1, C))
    return inv_traj, eq_traj


# speedup vs baseline: 1.2543x; 1.2543x over previous
import functools

import jax
import jax.numpy as jnp
import numpy as np
from jax.experimental import pallas as pl
from jax.experimental.pallas import tpu as pltpu

N = 4096
DIN = 128
H = 128
C = 16
L = 8

WB = 16
SGRID = N // WB
BN = 512

ROW_S = 32
ROW_L = 128

TINY = float(np.finfo(np.float32).tiny)

_FOLDED_KEYS = (
    (0x6D3E048F, 0x1022172D),
    (0x03D7B32D, 0xADD083F4),
    (0x92FB20EA, 0x0F38D913),
    (0xBAD56946, 0x354BA891),
    (0xB013AEE3, 0xC34EDDF6),
    (0xA4D91A96, 0x3122544E),
    (0xA506C508, 0xB6207291),
)


def _u32(v):
    return jnp.uint32(v & 0xFFFFFFFF)


def _threefry_bits(k0, k1, x1):
    ks0 = _u32(k0)
    ks1 = _u32(k1)
    ks2 = ks0 ^ ks1 ^ _u32(0x1BD11BDA)
    ks = (ks0, ks1, ks2)
    rotations = ((13, 15, 26, 6), (17, 29, 16, 24))
    x0 = jnp.full_like(x1, ks0)
    x1 = x1 + ks1
    for i in range(5):
        for r in rotations[i % 2]:
            x0 = x0 + x1
            x1 = (x1 << r) | (x1 >> (32 - r))
            x1 = x0 ^ x1
        x0 = x0 + ks[(i + 1) % 3]
        x1 = x1 + ks[(i + 2) % 3] + _u32(i + 1)
    return x0 ^ x1


def _gumbel_from_bits(bits):
    fb = (bits >> 9) | _u32(0x3F800000)
    f = jax.lax.bitcast_convert_type(fb, jnp.float32) - jnp.float32(1.0)
    one_minus_tiny = jnp.float32(np.float32(1.0) - np.float32(TINY))
    u = jnp.maximum(jnp.float32(TINY), f * one_minus_tiny + jnp.float32(TINY))
    return -jnp.log(-jnp.log(u))


def _row_dmas(cur_ref, step_idx, slot, prob_ref, buf_ref, psem,
              feat, inv_ref, eq_ref, ibuf_ref, ebuf_ref, isem, esem):
    copies = []
    for k in range(WB):
        r = cur_ref[step_idx * WB + k]
        copies.append(pltpu.make_async_copy(
            prob_ref.at[r], buf_ref.at[slot, k], psem.at[slot, k]))
        if feat:
            copies.append(pltpu.make_async_copy(
                inv_ref.at[r], ibuf_ref.at[slot, k], isem.at[slot, k]))
            copies.append(pltpu.make_async_copy(
                eq_ref.at[r], ebuf_ref.at[slot, k], esem.at[slot, k]))
    return copies


def _sample_body(k0, k1, feat, cur_ref, *refs):
    if feat:
        (prob_ref, inv_ref, eq_ref, nxt_ref, invcol_ref, eqcol_ref,
         buf_ref, ibuf_ref, ebuf_ref, psem, isem, esem) = refs
    else:
        prob_ref, nxt_ref, buf_ref, psem = refs
        inv_ref = eq_ref = ibuf_ref = ebuf_ref = isem = esem = None

    i = pl.program_id(0)

    @pl.when(i == 0)
    def _():
        for c in _row_dmas(cur_ref, 0, 0, prob_ref, buf_ref, psem,
                           feat, inv_ref, eq_ref, ibuf_ref, ebuf_ref,
                           isem, esem):
            c.start()

    @pl.when(i + 1 < SGRID)
    def _():
        for c in _row_dmas(cur_ref, i + 1, (i + 1) % 2, prob_ref, buf_ref,
                           psem, feat, inv_ref, eq_ref, ibuf_ref, ebuf_ref,
                           isem, esem):
            c.start()

    slot = i % 2
    for c in _row_dmas(cur_ref, i, slot, prob_ref, buf_ref, psem,
                       feat, inv_ref, eq_ref, ibuf_ref, ebuf_ref,
                       isem, esem):
        c.wait()

    flat = (jax.lax.broadcasted_iota(jnp.uint32, (WB, N), 0) * jnp.uint32(N)
            + jax.lax.broadcasted_iota(jnp.uint32, (WB, N), 1))
    t = jnp.uint32(i * (WB * N)) + flat
    g = _gumbel_from_bits(_threefry_bits(k0, k1, t))

    rows = buf_ref[slot]
    s = jnp.sum(rows, axis=1, keepdims=True)
    z = jnp.log(rows / s + jnp.float32(1e-9)) + g
    m = jnp.max(z, axis=1, keepdims=True)
    col = jax.lax.broadcasted_iota(jnp.int32, (WB, N), 1)
    idx = jnp.min(jnp.where(z >= m, col, jnp.int32(N)),
                  axis=1, keepdims=True)
    nxt_ref[0] = idx

    if feat:
        invcol_ref[...] = ibuf_ref[slot, :, 0, :]
        eqcol_ref[...] = ebuf_ref[slot, :, 0, :]


def _make_sample_call(step):
    k0, k1 = _FOLDED_KEYS[step]
    feat = step > 0
    body = functools.partial(_sample_body, k0, k1, feat)
    any_spec = pl.BlockSpec(memory_space=pl.ANY)
    in_specs = [any_spec]
    out_specs = [pl.BlockSpec((1, WB, 1), lambda i, c: (i, 0, 0))]
    out_shape = [jax.ShapeDtypeStruct((SGRID, WB, 1), jnp.int32)]
    scratch = [
        pltpu.VMEM((2, WB, N), jnp.float32),
    ]
    if feat:
        in_specs += [any_spec, any_spec]
        out_specs += [
            pl.BlockSpec((WB, DIN), lambda i, c: (i, 0)),
            pl.BlockSpec((WB, 3), lambda i, c: (i, 0)),
        ]
        out_shape += [
            jax.ShapeDtypeStruct((N, DIN), jnp.float32),
            jax.ShapeDtypeStruct((N, 3), jnp.float32),
        ]
        scratch += [
            pltpu.VMEM((2, WB, 1, DIN), jnp.float32),
            pltpu.VMEM((2, WB, 1, 3), jnp.float32),
        ]
    scratch += [pltpu.SemaphoreType.DMA((2, WB))]
    if feat:
        scratch += [pltpu.SemaphoreType.DMA((2, WB)),
                    pltpu.SemaphoreType.DMA((2, WB))]
    grid_spec = pltpu.PrefetchScalarGridSpec(
        num_scalar_prefetch=1,
        grid=(SGRID,),
        in_specs=in_specs,
        out_specs=out_specs,
        scratch_shapes=scratch,
    )
    return pl.pallas_call(
        body,
        grid_spec=grid_spec,
        out_shape=out_shape,
        compiler_params=pltpu.CompilerParams(
            dimension_semantics=("arbitrary",)),
    )


def _gather_body(cur_ref, inv_ref, eq_ref, invcol_ref, eqcol_ref,
                 ibuf_ref, ebuf_ref, isem, esem):
    i = pl.program_id(0)

    def dmas(step_idx, slot):
        copies = []
        for k in range(WB):
            r = cur_ref[step_idx * WB + k]
            copies.append(pltpu.make_async_copy(
                inv_ref.at[r], ibuf_ref.at[slot, k], isem.at[slot, k]))
            copies.append(pltpu.make_async_copy(
                eq_ref.at[r], ebuf_ref.at[slot, k], esem.at[slot, k]))
        return copies

    @pl.when(i == 0)
    def _():
        for c in dmas(0, 0):
            c.start()

    @pl.when(i + 1 < SGRID)
    def _():
        for c in dmas(i + 1, (i + 1) % 2):
            c.start()

    slot = i % 2
    for c in dmas(i, slot):
        c.wait()
    invcol_ref[...] = ibuf_ref[slot, :, 0, :]
    eqcol_ref[...] = ebuf_ref[slot, :, 0, :]


def _make_gather_call():
    any_spec = pl.BlockSpec(memory_space=pl.ANY)
    grid_spec = pltpu.PrefetchScalarGridSpec(
        num_scalar_prefetch=1,
        grid=(SGRID,),
        in_specs=[any_spec, any_spec],
        out_specs=[
            pl.BlockSpec((WB, DIN), lambda i, c: (i, 0)),
            pl.BlockSpec((WB, 3), lambda i, c: (i, 0)),
        ],
        scratch_shapes=[
            pltpu.VMEM((2, WB, 1, DIN), jnp.float32),
            pltpu.VMEM((2, WB, 1, 3), jnp.float32),
            pltpu.SemaphoreType.DMA((2, WB)),
            pltpu.SemaphoreType.DMA((2, WB)),
        ],
    )
    return pl.pallas_call(
        _gather_body,
        grid_spec=grid_spec,
        out_shape=[
            jax.ShapeDtypeStruct((N, DIN), jnp.float32),
            jax.ShapeDtypeStruct((N, 3), jnp.float32),
        ],
        compiler_params=pltpu.CompilerParams(
            dimension_semantics=("arbitrary",)),
    )


def _recur_body(*refs):
    inv_refs = refs[0:L]
    eq_refs = refs[L:2 * L]
    w_in_ref, w_h_ref, b_ref, w_gate_ref, w_mix_ref = refs[2 * L:2 * L + 5]
    inv_traj_ref, eq_traj_ref = refs[2 * L + 5:]

    pooled_inv = inv_refs[0][...]
    for r in inv_refs[1:]:
        pooled_inv = pooled_inv + r[...]
    pooled_inv = pooled_inv * jnp.float32(1.0 / L)

    pooled_eq = eq_refs[0][...]
    for r in eq_refs[1:]:
        pooled_eq = pooled_eq + r[...]
    pooled_eq = pooled_eq * jnp.float32(1.0 / L)

    w_in = w_in_ref[...]
    w_h = w_h_ref[...]
    b = b_ref[...]
    w_gate = w_gate_ref[...]
    w_mix = w_mix_ref[...]

    a = jnp.dot(pooled_inv, w_in, preferred_element_type=jnp.float32) + b
    src = [pooled_eq[:, d:d + 1] * w_mix for d in range(3)]

    inv_h = jnp.zeros((BN, H), jnp.float32)
    eq_h = [jnp.zeros((BN, C), jnp.float32) for _ in range(3)]
    for step in range(L):
        inv_h = jnp.tanh(
            a + jnp.dot(inv_h, w_h, preferred_element_type=jnp.float32))
        gate = jax.nn.sigmoid(
            jnp.dot(inv_h, w_gate, preferred_element_type=jnp.float32))
        inv_traj_ref[step] = inv_h
        for d in range(3):
            eq_h[d] = eq_h[d] * gate + src[d]
            eq_traj_ref[step, :, d, :] = eq_h[d]


def _recur_call():
    nb = N // BN
    full2 = lambda shape: pl.BlockSpec(shape, lambda i: (0, 0))
    return pl.pallas_call(
        _recur_body,
        grid=(nb,),
        in_specs=[pl.BlockSpec((BN, DIN), lambda i: (i, 0)) for _ in range(L)]
        + [pl.BlockSpec((BN, 3), lambda i: (i, 0)) for _ in range(L)]
        + [
            full2((DIN, H)),
            full2((H, H)),
            full2((1, H)),
            full2((H, C)),
            full2((1, C)),
        ],
        out_specs=[
            pl.BlockSpec((L, BN, H), lambda i: (0, i, 0)),
            pl.BlockSpec((L, BN, 3, C), lambda i: (0, i, 0, 0)),
        ],
        out_shape=[
            jax.ShapeDtypeStruct((L, N, H), jnp.float32),
            jax.ShapeDtypeStruct((L, N, 3, C), jnp.float32),
        ],
    )


@jax.jit
def kernel(probability, invariant_input, equivariant_input, W_in, W_h, b,
           W_gate, w_mix):
    cur = jnp.arange(N, dtype=jnp.int32)
    inv3 = invariant_input.reshape(N, 1, DIN)
    eq3 = equivariant_input.reshape(N, 1, 3)

    inv_cols = [invariant_input]
    eq_cols = [equivariant_input]

    [nxt] = _make_sample_call(0)(cur, probability)
    cur = nxt.reshape(N)
    for step in range(1, L - 1):
        nxt, invcol, eqcol = _make_sample_call(step)(cur, probability, inv3, eq3)
        inv_cols.append(invcol)
        eq_cols.append(eqcol)
        cur = nxt.reshape(N)
    invcol, eqcol = _make_gather_call()(cur, inv3, eq3)
    inv_cols.append(invcol)
    eq_cols.append(eqcol)

    inv_traj, eq_traj = _recur_call()(
        *inv_cols, *eq_cols, W_in, W_h, b.reshape(1, H), W_gate,
        w_mix.reshape(1, C))
    return inv_traj, eq_traj
